# baseline (speedup 1.0000x reference)
import jax
import jax.numpy as jnp
from jax import lax
from jax.experimental import pallas as pl
from jax.experimental.pallas import tpu as pltpu

FEATS = 1024
HID = 16384
NTOK = 4096
KTOP = 32

CH = 16
NCB = 1024 // CH

BM = 128
BH = 1024
NJ = HID // BH

BM2 = 512
BK = 4096

def _sort_key(af):
    bits = lax.bitcast_convert_type(af, jnp.int32)
    m = (bits >> 31) & jnp.int32(0x7FFFFFFF)
    return bits ^ m


def _enc_kernel(x_ref, we_ref, be_ref, bd_ref, f_ref, u_scr, mk_scr):
    j = pl.program_id(1)
    xbar = x_ref[...] - bd_ref[...]
    a = lax.dot_general(xbar, we_ref[...], (((1,), (1,)), ((), ())),
                        preferred_element_type=jnp.float32)
    a = a + be_ref[...]
    u = _sort_key(a)
    u_scr[j] = u
    mk_scr[j] = jnp.max(u.reshape(BM, NCB, CH), axis=-1)

    @pl.when(j == NJ - 1)
    def _():
        mkey = lax.bitcast_convert_type(
            mk_scr[...], jnp.uint32) ^ jnp.uint32(0x80000000)

        def rbody(t, prefix):
            bit = lax.shift_right_logical(
                jnp.uint32(0x80000000), t.astype(jnp.uint32))
            trial = prefix | bit
            cnt = jnp.sum((mkey >= trial).astype(jnp.int32),
                          axis=(0, 2), keepdims=True)
            return jnp.where(cnt >= KTOP, trial, prefix)

        t0u = lax.fori_loop(0, 32, rbody, jnp.zeros((1, BM, 1), jnp.uint32))
        t0 = lax.bitcast_convert_type(t0u ^ jnp.uint32(0x80000000), jnp.int32)

        ukey = u_scr[...]
        cnt0 = jnp.sum((ukey >= t0).astype(jnp.int32),
                       axis=(0, 2), keepdims=True)

        def wcond(state):
            _, cnt = state
            return jnp.max(cnt) > KTOP

        def wbody(state):
            thr, cnt = state
            over = cnt > KTOP
            sel_min = jnp.min(jnp.where(ukey >= thr, ukey, jnp.int32(0x7FFFFFFF)),
                              axis=(0, 2), keepdims=True)
            thr = jnp.where(over, sel_min + jnp.int32(1), thr)
            cnt = cnt - over.astype(jnp.int32)
            return thr, cnt

        thr, _ = lax.while_loop(wcond, wbody, (t0, cnt0))
        thr2 = jnp.maximum(thr[0], jnp.int32(0))
        for jj in range(NJ):
            uj = u_scr[jj]
            f_ref[:, jj * BH:(jj + 1) * BH] = jnp.where(
                uj >= thr2, lax.bitcast_convert_type(uj, jnp.float32), 0.0)


def _dec_kernel(f_ref, wd_ref, bd_ref, xhat_ref):
    kblk = pl.program_id(1)
    part = lax.dot_general(f_ref[...], wd_ref[...], (((1,), (1,)), ((), ())),
                           preferred_element_type=jnp.float32)

    @pl.when(kblk == 0)
    def _():
        xhat_ref[...] = bd_ref[...] + part

    @pl.when(kblk != 0)
    def _():
        xhat_ref[...] = xhat_ref[...] + part


@jax.jit
def _run(x, W_enc, b_enc, W_dec, b_dec):
    be2 = b_enc.reshape(1, HID)
    bd2 = b_dec.reshape(1, FEATS)

    f = pl.pallas_call(
        _enc_kernel,
        grid=(NTOK // BM, NJ),
        in_specs=[
            pl.BlockSpec((BM, FEATS), lambda i, j: (i, 0)),
            pl.BlockSpec((BH, FEATS), lambda i, j: (j, 0)),
            pl.BlockSpec((1, BH), lambda i, j: (0, j)),
            pl.BlockSpec((1, FEATS), lambda i, j: (0, 0)),
        ],
        out_specs=pl.BlockSpec((BM, HID), lambda i, j: (i, 0)),
        scratch_shapes=[
            pltpu.VMEM((NJ, BM, BH), jnp.int32),
            pltpu.VMEM((NJ, BM, NCB), jnp.int32),
        ],
        out_shape=jax.ShapeDtypeStruct((NTOK, HID), jnp.float32),
    )(x, W_enc, be2, bd2)

    xhat = pl.pallas_call(
        _dec_kernel,
        grid=(NTOK // BM2, HID // BK),
        in_specs=[
            pl.BlockSpec((BM2, BK), lambda i, k: (i, k)),
            pl.BlockSpec((FEATS, BK), lambda i, k: (0, k)),
            pl.BlockSpec((1, FEATS), lambda i, k: (0, 0)),
        ],
        out_specs=pl.BlockSpec((BM2, FEATS), lambda i, k: (i, 0)),
        out_shape=jax.ShapeDtypeStruct((NTOK, FEATS), jnp.float32),
    )(f, W_dec, bd2)

    return xhat, f


def kernel(x, W_enc, b_enc, W_dec, b_dec, K):
    return _run(x, W_enc, b_enc, W_dec, b_dec)


# speedup vs baseline: 1.8241x; 1.8241x over previous
import jax
import jax.numpy as jnp
from jax import lax
from jax.experimental import pallas as pl
from jax.experimental.pallas import tpu as pltpu

FEATS = 1024
HID = 16384
NTOK = 4096
KTOP = 32

NCB = 1024

BM = 128
BH = 1024
NJ = HID // BH

BM2 = 512
BK = 4096

def _sort_key(af):
    bits = lax.bitcast_convert_type(af, jnp.int32)
    m = (bits >> 31) & jnp.int32(0x7FFFFFFF)
    return bits ^ m


def _enc_kernel(x_ref, we_ref, be_ref, bd_ref, f_ref, u_scr, mk_scr):
    j = pl.program_id(1)
    xbar = x_ref[...] - bd_ref[...]
    a = lax.dot_general(xbar, we_ref[...], (((1,), (1,)), ((), ())),
                        preferred_element_type=jnp.float32)
    a = a + be_ref[...]
    u = _sort_key(a)
    u_scr[j] = u

    @pl.when(j == 0)
    def _():
        mk_scr[...] = u

    @pl.when(j != 0)
    def _():
        mk_scr[...] = jnp.maximum(mk_scr[...], u)

    @pl.when(j == NJ - 1)
    def _():
        mkey = lax.shift_right_logical(
            lax.bitcast_convert_type(mk_scr[...], jnp.uint32)
            ^ jnp.uint32(0x80000000), jnp.uint32(16))

        def rbody(t, prefix):
            bit = lax.shift_right_logical(
                jnp.uint32(0x8000), t.astype(jnp.uint32))
            trial = prefix | bit
            cnt = jnp.sum((mkey >= trial).astype(jnp.int32),
                          axis=1, keepdims=True)
            return jnp.where(cnt >= KTOP, trial, prefix)

        t0u = lax.fori_loop(0, 16, rbody, jnp.zeros((BM, 1), jnp.uint32))
        t0 = lax.bitcast_convert_type(
            (t0u << jnp.uint32(16)) ^ jnp.uint32(0x80000000), jnp.int32)
        t0 = t0.reshape(1, BM, 1)

        ukey = u_scr[...]
        cnt0 = jnp.sum((ukey >= t0).astype(jnp.int32),
                       axis=(0, 2), keepdims=True)

        def wcond(state):
            _, cnt = state
            return jnp.max(cnt) > KTOP

        def wbody(state):
            thr, cnt = state
            over = cnt > KTOP
            sel_min = jnp.min(jnp.where(ukey >= thr, ukey, jnp.int32(0x7FFFFFFF)),
                              axis=(0, 2), keepdims=True)
            thr = jnp.where(over, sel_min + jnp.int32(1), thr)
            cnt = cnt - over.astype(jnp.int32)
            return thr, cnt

        thr, _ = lax.while_loop(wcond, wbody, (t0, cnt0))
        thr2 = jnp.maximum(thr[0], jnp.int32(0))
        for jj in range(NJ):
            uj = u_scr[jj]
            f_ref[:, jj * BH:(jj + 1) * BH] = jnp.where(
                uj >= thr2, lax.bitcast_convert_type(uj, jnp.float32), 0.0)


def _dec_kernel(f_ref, wd_ref, bd_ref, xhat_ref):
    kblk = pl.program_id(1)
    part = lax.dot_general(f_ref[...], wd_ref[...], (((1,), (1,)), ((), ())),
                           preferred_element_type=jnp.float32)

    @pl.when(kblk == 0)
    def _():
        xhat_ref[...] = bd_ref[...] + part

    @pl.when(kblk != 0)
    def _():
        xhat_ref[...] = xhat_ref[...] + part


@jax.jit
def _run(x, W_enc, b_enc, W_dec, b_dec):
    be2 = b_enc.reshape(1, HID)
    bd2 = b_dec.reshape(1, FEATS)

    f = pl.pallas_call(
        _enc_kernel,
        grid=(NTOK // BM, NJ),
        in_specs=[
            pl.BlockSpec((BM, FEATS), lambda i, j: (i, 0)),
            pl.BlockSpec((BH, FEATS), lambda i, j: (j, 0)),
            pl.BlockSpec((1, BH), lambda i, j: (0, j)),
            pl.BlockSpec((1, FEATS), lambda i, j: (0, 0)),
        ],
        out_specs=pl.BlockSpec((BM, HID), lambda i, j: (i, 0)),
        scratch_shapes=[
            pltpu.VMEM((NJ, BM, BH), jnp.int32),
            pltpu.VMEM((BM, NCB), jnp.int32),
        ],
        out_shape=jax.ShapeDtypeStruct((NTOK, HID), jnp.float32),
    )(x, W_enc, be2, bd2)

    xhat = pl.pallas_call(
        _dec_kernel,
        grid=(NTOK // BM2, HID // BK),
        in_specs=[
            pl.BlockSpec((BM2, BK), lambda i, k: (i, k)),
            pl.BlockSpec((FEATS, BK), lambda i, k: (0, k)),
            pl.BlockSpec((1, FEATS), lambda i, k: (0, 0)),
        ],
        out_specs=pl.BlockSpec((BM2, FEATS), lambda i, k: (i, 0)),
        out_shape=jax.ShapeDtypeStruct((NTOK, FEATS), jnp.float32),
    )(f, W_dec, bd2)

    return xhat, f


def kernel(x, W_enc, b_enc, W_dec, b_dec, K):
    return _run(x, W_enc, b_enc, W_dec, b_dec)


# speedup vs baseline: 2.0912x; 1.1465x over previous
import jax
import jax.numpy as jnp
from jax import lax
from jax.experimental import pallas as pl
from jax.experimental.pallas import tpu as pltpu

FEATS = 1024
HID = 16384
NTOK = 4096
KTOP = 32

NCB = 2048

BM = 128
BH = 2048
NJ = HID // BH

BM2 = 512
BK = 4096

def _sort_key(af):
    bits = lax.bitcast_convert_type(af, jnp.int32)
    m = (bits >> 31) & jnp.int32(0x7FFFFFFF)
    return bits ^ m


def _enc_kernel(x_ref, we_ref, be_ref, bd_ref, f_ref, u_scr, mk_scr):
    j = pl.program_id(1)
    xbar = x_ref[...] - bd_ref[...]
    a = lax.dot_general(xbar, we_ref[...], (((1,), (1,)), ((), ())),
                        preferred_element_type=jnp.float32)
    a = a + be_ref[...]
    u = _sort_key(a)
    u_scr[j] = u

    @pl.when(j == 0)
    def _():
        mk_scr[...] = u

    @pl.when(j != 0)
    def _():
        mk_scr[...] = jnp.maximum(mk_scr[...], u)

    @pl.when(j == NJ - 1)
    def _():
        mkey = lax.shift_right_logical(
            lax.bitcast_convert_type(mk_scr[...], jnp.uint32)
            ^ jnp.uint32(0x80000000), jnp.uint32(8))

        def rbody(t, prefix):
            bit = lax.shift_right_logical(
                jnp.uint32(0x800000), t.astype(jnp.uint32))
            trial = prefix | bit
            cnt = jnp.sum((mkey >= trial).astype(jnp.int32),
                          axis=1, keepdims=True)
            return jnp.where(cnt >= KTOP, trial, prefix)

        t0u = lax.fori_loop(0, 24, rbody, jnp.zeros((BM, 1), jnp.uint32))
        t0 = lax.bitcast_convert_type(
            (t0u << jnp.uint32(8)) ^ jnp.uint32(0x80000000), jnp.int32)
        t0 = t0.reshape(1, BM, 1)

        ukey = u_scr[...]
        cnt0 = jnp.sum((ukey >= t0).astype(jnp.int32),
                       axis=(0, 2), keepdims=True)

        def wcond(state):
            _, cnt = state
            return jnp.max(cnt) > KTOP

        def wbody(state):
            thr, cnt = state
            over = cnt > KTOP
            sel_min = jnp.min(jnp.where(ukey >= thr, ukey, jnp.int32(0x7FFFFFFF)),
                              axis=(0, 2), keepdims=True)
            thr = jnp.where(over, sel_min + jnp.int32(1), thr)
            cnt = cnt - over.astype(jnp.int32)
            return thr, cnt

        thr, _ = lax.while_loop(wcond, wbody, (t0, cnt0))
        thr2 = jnp.maximum(thr[0], jnp.int32(0))
        for jj in range(NJ):
            uj = u_scr[jj]
            f_ref[:, jj * BH:(jj + 1) * BH] = jnp.where(
                uj >= thr2, lax.bitcast_convert_type(uj, jnp.float32), 0.0)


def _dec_kernel(f_ref, wd_ref, bd_ref, xhat_ref):
    kblk = pl.program_id(1)
    part = lax.dot_general(f_ref[...], wd_ref[...], (((1,), (1,)), ((), ())),
                           preferred_element_type=jnp.float32)

    @pl.when(kblk == 0)
    def _():
        xhat_ref[...] = bd_ref[...] + part

    @pl.when(kblk != 0)
    def _():
        xhat_ref[...] = xhat_ref[...] + part


@jax.jit
def _run(x, W_enc, b_enc, W_dec, b_dec):
    be2 = b_enc.reshape(1, HID)
    bd2 = b_dec.reshape(1, FEATS)

    f = pl.pallas_call(
        _enc_kernel,
        grid=(NTOK // BM, NJ),
        in_specs=[
            pl.BlockSpec((BM, FEATS), lambda i, j: (i, 0)),
            pl.BlockSpec((BH, FEATS), lambda i, j: (j, 0)),
            pl.BlockSpec((1, BH), lambda i, j: (0, j)),
            pl.BlockSpec((1, FEATS), lambda i, j: (0, 0)),
        ],
        out_specs=pl.BlockSpec((BM, HID), lambda i, j: (i, 0)),
        scratch_shapes=[
            pltpu.VMEM((NJ, BM, BH), jnp.int32),
            pltpu.VMEM((BM, NCB), jnp.int32),
        ],
        out_shape=jax.ShapeDtypeStruct((NTOK, HID), jnp.float32),
    )(x, W_enc, be2, bd2)

    xhat = pl.pallas_call(
        _dec_kernel,
        grid=(NTOK // BM2, HID // BK),
        in_specs=[
            pl.BlockSpec((BM2, BK), lambda i, k: (i, k)),
            pl.BlockSpec((FEATS, BK), lambda i, k: (0, k)),
            pl.BlockSpec((1, FEATS), lambda i, k: (0, 0)),
        ],
        out_specs=pl.BlockSpec((BM2, FEATS), lambda i, k: (i, 0)),
        out_shape=jax.ShapeDtypeStruct((NTOK, FEATS), jnp.float32),
    )(f, W_dec, bd2)

    return xhat, f


def kernel(x, W_enc, b_enc, W_dec, b_dec, K):
    return _run(x, W_enc, b_enc, W_dec, b_dec)


# speedup vs baseline: 2.2568x; 1.0792x over previous
import jax
import jax.numpy as jnp
from jax import lax
from jax.experimental import pallas as pl
from jax.experimental.pallas import tpu as pltpu

FEATS = 1024
HID = 16384
NTOK = 4096
KTOP = 32

NCB = 512

BM = 256
BH = 512
NJ = HID // BH

BM2 = 512
BK = 4096

def _sort_key(af):
    bits = lax.bitcast_convert_type(af, jnp.int32)
    m = (bits >> 31) & jnp.int32(0x7FFFFFFF)
    return bits ^ m


def _enc_kernel(x_ref, we_ref, be_ref, bd_ref, f_ref, mk_scr):
    j = pl.program_id(1)
    xbar = x_ref[...] - bd_ref[...]
    a = lax.dot_general(xbar, we_ref[...], (((1,), (1,)), ((), ())),
                        preferred_element_type=jnp.float32)
    a = a + be_ref[...]
    u = _sort_key(a)
    f_ref[:, pl.ds(pl.multiple_of(j * BH, BH), BH)] = lax.bitcast_convert_type(
        u, jnp.float32)

    @pl.when(j == 0)
    def _():
        mk_scr[...] = u

    @pl.when(j != 0)
    def _():
        mk_scr[...] = jnp.maximum(mk_scr[...], u)

    @pl.when(j == NJ - 1)
    def _():
        mkey = lax.shift_right_logical(
            lax.bitcast_convert_type(mk_scr[...], jnp.uint32)
            ^ jnp.uint32(0x80000000), jnp.uint32(8))

        def rbody(t, prefix):
            bit = lax.shift_right_logical(
                jnp.uint32(0x800000), t.astype(jnp.uint32))
            trial = prefix | bit
            cnt = jnp.sum((mkey >= trial).astype(jnp.int32),
                          axis=1, keepdims=True)
            return jnp.where(cnt >= KTOP, trial, prefix)

        t0u = lax.fori_loop(0, 24, rbody, jnp.zeros((BM, 1), jnp.uint32))
        t0 = lax.bitcast_convert_type(
            (t0u << jnp.uint32(8)) ^ jnp.uint32(0x80000000), jnp.int32)

        ukey = lax.bitcast_convert_type(f_ref[...], jnp.int32)
        cnt0 = jnp.sum((ukey >= t0).astype(jnp.int32),
                       axis=1, keepdims=True)

        def wcond(state):
            _, cnt = state
            return jnp.max(cnt) > KTOP

        def wbody(state):
            thr, cnt = state
            over = cnt > KTOP
            sel_min = jnp.min(jnp.where(ukey >= thr, ukey,
                                        jnp.int32(0x7FFFFFFF)),
                              axis=1, keepdims=True)
            thr = jnp.where(over, sel_min + jnp.int32(1), thr)
            cnt = cnt - over.astype(jnp.int32)
            return thr, cnt

        thr, _ = lax.while_loop(wcond, wbody, (t0, cnt0))
        thr2 = jnp.maximum(thr, jnp.int32(0))
        for jj in range(NJ):
            uj = lax.bitcast_convert_type(
                f_ref[:, jj * BH:(jj + 1) * BH], jnp.int32)
            f_ref[:, jj * BH:(jj + 1) * BH] = jnp.where(
                uj >= thr2, lax.bitcast_convert_type(uj, jnp.float32), 0.0)


def _dec_kernel(f_ref, wd_ref, bd_ref, xhat_ref):
    kblk = pl.program_id(1)
    part = lax.dot_general(f_ref[...], wd_ref[...], (((1,), (1,)), ((), ())),
                           preferred_element_type=jnp.float32)

    @pl.when(kblk == 0)
    def _():
        xhat_ref[...] = bd_ref[...] + part

    @pl.when(kblk != 0)
    def _():
        xhat_ref[...] = xhat_ref[...] + part


@jax.jit
def _run(x, W_enc, b_enc, W_dec, b_dec):
    be2 = b_enc.reshape(1, HID)
    bd2 = b_dec.reshape(1, FEATS)

    f = pl.pallas_call(
        _enc_kernel,
        grid=(NTOK // BM, NJ),
        in_specs=[
            pl.BlockSpec((BM, FEATS), lambda i, j: (i, 0)),
            pl.BlockSpec((BH, FEATS), lambda i, j: (j, 0)),
            pl.BlockSpec((1, BH), lambda i, j: (0, j)),
            pl.BlockSpec((1, FEATS), lambda i, j: (0, 0)),
        ],
        out_specs=pl.BlockSpec((BM, HID), lambda i, j: (i, 0)),
        scratch_shapes=[
            pltpu.VMEM((BM, NCB), jnp.int32),
        ],
        out_shape=jax.ShapeDtypeStruct((NTOK, HID), jnp.float32),
    )(x, W_enc, be2, bd2)

    xhat = pl.pallas_call(
        _dec_kernel,
        grid=(NTOK // BM2, HID // BK),
        in_specs=[
            pl.BlockSpec((BM2, BK), lambda i, k: (i, k)),
            pl.BlockSpec((FEATS, BK), lambda i, k: (0, k)),
            pl.BlockSpec((1, FEATS), lambda i, k: (0, 0)),
        ],
        out_specs=pl.BlockSpec((BM2, FEATS), lambda i, k: (i, 0)),
        out_shape=jax.ShapeDtypeStruct((NTOK, FEATS), jnp.float32),
    )(f, W_dec, bd2)

    return xhat, f


def kernel(x, W_enc, b_enc, W_dec, b_dec, K):
    return _run(x, W_enc, b_enc, W_dec, b_dec)


# speedup vs baseline: 3.1763x; 1.4074x over previous
import jax
import jax.numpy as jnp
from jax import lax
from jax.experimental import pallas as pl
from jax.experimental.pallas import tpu as pltpu

FEATS = 1024
HID = 16384
NTOK = 4096
KTOP = 32

NCB = 512

BM = 512
BH = 4096
NJ = HID // BH
BMB = 128
NSL = HID // NCB

BM2 = 512
BK = 4096

def _sort_key(af):
    bits = lax.bitcast_convert_type(af, jnp.int32)
    m = (bits >> 31) & jnp.int32(0x7FFFFFFF)
    return bits ^ m


def _mm_kernel(x_ref, we_ref, be_ref, bd_ref, k_ref):
    xbar = x_ref[...] - bd_ref[...]
    a = lax.dot_general(xbar, we_ref[...], (((1,), (1,)), ((), ())),
                        preferred_element_type=jnp.float32)
    k_ref[...] = _sort_key(a + be_ref[...])


def _fix_kernel(k_ref, f_ref):
    ukey = k_ref[...]
    mk = ukey[:, 0:NCB]
    for jj in range(1, NSL):
        mk = jnp.maximum(mk, ukey[:, jj * NCB:(jj + 1) * NCB])
    mkey = lax.shift_right_logical(
        lax.bitcast_convert_type(mk, jnp.uint32)
        ^ jnp.uint32(0x80000000), jnp.uint32(8))

    def rbody(t, prefix):
        bit = lax.shift_right_logical(
            jnp.uint32(0x800000), t.astype(jnp.uint32))
        trial = prefix | bit
        cnt = jnp.sum((mkey >= trial).astype(jnp.int32),
                      axis=1, keepdims=True)
        return jnp.where(cnt >= KTOP, trial, prefix)

    t0u = lax.fori_loop(0, 24, rbody, jnp.zeros((BMB, 1), jnp.uint32))
    t0 = lax.bitcast_convert_type(
        (t0u << jnp.uint32(8)) ^ jnp.uint32(0x80000000), jnp.int32)

    cnt0 = jnp.sum((ukey >= t0).astype(jnp.int32), axis=1, keepdims=True)

    def wcond(state):
        _, cnt = state
        return jnp.max(cnt) > KTOP

    def wbody(state):
        thr, cnt = state
        over = cnt > KTOP
        sel_min = jnp.min(jnp.where(ukey >= thr, ukey, jnp.int32(0x7FFFFFFF)),
                          axis=1, keepdims=True)
        thr = jnp.where(over, sel_min + jnp.int32(1), thr)
        cnt = cnt - over.astype(jnp.int32)
        return thr, cnt

    thr, _ = lax.while_loop(wcond, wbody, (t0, cnt0))
    thr2 = jnp.maximum(thr, jnp.int32(0))
    f_ref[...] = jnp.where(ukey >= thr2,
                           lax.bitcast_convert_type(ukey, jnp.float32), 0.0)


def _dec_kernel(f_ref, wd_ref, bd_ref, xhat_ref):
    kblk = pl.program_id(1)
    part = lax.dot_general(f_ref[...], wd_ref[...], (((1,), (1,)), ((), ())),
                           preferred_element_type=jnp.float32)

    @pl.when(kblk == 0)
    def _():
        xhat_ref[...] = bd_ref[...] + part

    @pl.when(kblk != 0)
    def _():
        xhat_ref[...] = xhat_ref[...] + part


@jax.jit
def _run(x, W_enc, b_enc, W_dec, b_dec):
    be2 = b_enc.reshape(1, HID)
    bd2 = b_dec.reshape(1, FEATS)

    keys = pl.pallas_call(
        _mm_kernel,
        grid=(NTOK // BM, NJ),
        in_specs=[
            pl.BlockSpec((BM, FEATS), lambda i, j: (i, 0)),
            pl.BlockSpec((BH, FEATS), lambda i, j: (j, 0)),
            pl.BlockSpec((1, BH), lambda i, j: (0, j)),
            pl.BlockSpec((1, FEATS), lambda i, j: (0, 0)),
        ],
        out_specs=pl.BlockSpec((BM, BH), lambda i, j: (i, j)),
        out_shape=jax.ShapeDtypeStruct((NTOK, HID), jnp.int32),
    )(x, W_enc, be2, bd2)

    f = pl.pallas_call(
        _fix_kernel,
        grid=(NTOK // BMB,),
        in_specs=[pl.BlockSpec((BMB, HID), lambda i: (i, 0))],
        out_specs=pl.BlockSpec((BMB, HID), lambda i: (i, 0)),
        out_shape=jax.ShapeDtypeStruct((NTOK, HID), jnp.float32),
    )(keys)

    xhat = pl.pallas_call(
        _dec_kernel,
        grid=(NTOK // BM2, HID // BK),
        in_specs=[
            pl.BlockSpec((BM2, BK), lambda i, k: (i, k)),
            pl.BlockSpec((FEATS, BK), lambda i, k: (0, k)),
            pl.BlockSpec((1, FEATS), lambda i, k: (0, 0)),
        ],
        out_specs=pl.BlockSpec((BM2, FEATS), lambda i, k: (i, 0)),
        out_shape=jax.ShapeDtypeStruct((NTOK, FEATS), jnp.float32),
    )(f, W_dec, bd2)

    return xhat, f


def kernel(x, W_enc, b_enc, W_dec, b_dec, K):
    return _run(x, W_enc, b_enc, W_dec, b_dec)


# speedup vs baseline: 3.8627x; 1.2161x over previous
import jax
import jax.numpy as jnp
from jax import lax
from jax.experimental import pallas as pl
from jax.experimental.pallas import tpu as pltpu

FEATS = 1024
HID = 16384
NTOK = 4096
KTOP = 32

NCB = 512

BM = 2048
BH = 1024
NJ = HID // BH
BMB = 128
NSL = HID // NCB

BM2 = 2048
BK = 1024

def _sort_key(af):
    bits = lax.bitcast_convert_type(af, jnp.int32)
    m = (bits >> 31) & jnp.int32(0x7FFFFFFF)
    return bits ^ m


def _mm_kernel(x_ref, we_ref, be_ref, bd_ref, k_ref):
    xbar = x_ref[...] - bd_ref[...]
    a = lax.dot_general(xbar, we_ref[...], (((1,), (1,)), ((), ())),
                        preferred_element_type=jnp.float32)
    k_ref[...] = _sort_key(a + be_ref[...])


def _fix_kernel(k_ref, f_ref):
    ukey = k_ref[...]
    mk = ukey[:, 0:NCB]
    for jj in range(1, NSL):
        mk = jnp.maximum(mk, ukey[:, jj * NCB:(jj + 1) * NCB])
    mkey = lax.shift_right_logical(
        lax.bitcast_convert_type(mk, jnp.uint32)
        ^ jnp.uint32(0x80000000), jnp.uint32(8))

    def rbody(t, prefix):
        bit = lax.shift_right_logical(
            jnp.uint32(0x800000), t.astype(jnp.uint32))
        trial = prefix | bit
        cnt = jnp.sum((mkey >= trial).astype(jnp.int32),
                      axis=1, keepdims=True)
        return jnp.where(cnt >= KTOP, trial, prefix)

    t0u = lax.fori_loop(0, 24, rbody, jnp.zeros((BMB, 1), jnp.uint32))
    t0 = lax.bitcast_convert_type(
        (t0u << jnp.uint32(8)) ^ jnp.uint32(0x80000000), jnp.int32)

    cnt0 = jnp.sum((ukey >= t0).astype(jnp.int32), axis=1, keepdims=True)

    def wcond(state):
        _, cnt = state
        return jnp.max(cnt) > KTOP

    def wbody(state):
        thr, cnt = state
        over = cnt > KTOP
        sel_min = jnp.min(jnp.where(ukey >= thr, ukey, jnp.int32(0x7FFFFFFF)),
                          axis=1, keepdims=True)
        thr = jnp.where(over, sel_min + jnp.int32(1), thr)
        cnt = cnt - over.astype(jnp.int32)
        return thr, cnt

    thr, _ = lax.while_loop(wcond, wbody, (t0, cnt0))
    thr2 = jnp.maximum(thr, jnp.int32(0))
    f_ref[...] = jnp.where(ukey >= thr2,
                           lax.bitcast_convert_type(ukey, jnp.float32), 0.0)


def _dec_kernel(f_ref, wd_ref, bd_ref, xhat_ref):
    kblk = pl.program_id(1)
    part = lax.dot_general(f_ref[...], wd_ref[...], (((1,), (1,)), ((), ())),
                           preferred_element_type=jnp.float32)

    @pl.when(kblk == 0)
    def _():
        xhat_ref[...] = bd_ref[...] + part

    @pl.when(kblk != 0)
    def _():
        xhat_ref[...] = xhat_ref[...] + part


@jax.jit
def _run(x, W_enc, b_enc, W_dec, b_dec):
    be2 = b_enc.reshape(1, HID)
    bd2 = b_dec.reshape(1, FEATS)

    keys = pl.pallas_call(
        _mm_kernel,
        grid=(NTOK // BM, NJ),
        in_specs=[
            pl.BlockSpec((BM, FEATS), lambda i, j: (i, 0)),
            pl.BlockSpec((BH, FEATS), lambda i, j: (j, 0)),
            pl.BlockSpec((1, BH), lambda i, j: (0, j)),
            pl.BlockSpec((1, FEATS), lambda i, j: (0, 0)),
        ],
        out_specs=pl.BlockSpec((BM, BH), lambda i, j: (i, j)),
        out_shape=jax.ShapeDtypeStruct((NTOK, HID), jnp.int32),
    )(x, W_enc, be2, bd2)

    f = pl.pallas_call(
        _fix_kernel,
        grid=(NTOK // BMB,),
        in_specs=[pl.BlockSpec((BMB, HID), lambda i: (i, 0))],
        out_specs=pl.BlockSpec((BMB, HID), lambda i: (i, 0)),
        out_shape=jax.ShapeDtypeStruct((NTOK, HID), jnp.float32),
    )(keys)

    xhat = pl.pallas_call(
        _dec_kernel,
        grid=(NTOK // BM2, HID // BK),
        in_specs=[
            pl.BlockSpec((BM2, BK), lambda i, k: (i, k)),
            pl.BlockSpec((FEATS, BK), lambda i, k: (0, k)),
            pl.BlockSpec((1, FEATS), lambda i, k: (0, 0)),
        ],
        out_specs=pl.BlockSpec((BM2, FEATS), lambda i, k: (i, 0)),
        out_shape=jax.ShapeDtypeStruct((NTOK, FEATS), jnp.float32),
    )(f, W_dec, bd2)

    return xhat, f


def kernel(x, W_enc, b_enc, W_dec, b_dec, K):
    return _run(x, W_enc, b_enc, W_dec, b_dec)
